# baseline (speedup 1.0000x reference)
import functools

import jax
import jax.numpy as jnp
from jax import lax
from jax.experimental import pallas as pl
from jax.experimental.pallas import tpu as pltpu
from jax.experimental.pallas import tpu_sc as plsc

N = 10000
E = 320000
C = 128

NC = 2
NS = 16
NW = NC * NS
EPW = E // NW
LANES = 16
DEG_STEPS = EPW // LANES

CHUNK = 64
NCHUNK = 160
EPT = NCHUNK * CHUNK
E_PAD = NW * EPT
BLK = 8
NBLK = NCHUNK // BLK
NBUF = 2
N_ACC = 10256
RPW = 640
RTAIL = N_ACC - NS * RPW

_mesh = plsc.VectorSubcoreMesh(core_axis_name="c", subcore_axis_name="s")


@functools.partial(
    pl.kernel,
    out_type=jax.ShapeDtypeStruct((NW, N), jnp.float32),
    mesh=_mesh,
    compiler_params=pltpu.CompilerParams(needs_layout_passes=False),
    scratch_types=[
        pltpu.VMEM((EPW,), jnp.int32),
        pltpu.VMEM((N,), jnp.float32),
    ],
)
def _deg_kernel(dst_hbm, out_hbm, dst_v, deg_v):
    wid = lax.axis_index("c") * NS + lax.axis_index("s")
    pltpu.sync_copy(dst_hbm.at[wid], dst_v)

    zeros16 = jnp.zeros((LANES,), jnp.float32)
    ones16 = jnp.ones((LANES,), jnp.float32)

    def _zero(i, carry):
        deg_v[pl.ds(i * LANES, LANES)] = zeros16
        return carry

    lax.fori_loop(0, N // LANES, _zero, 0, unroll=8)

    def _count(i, carry):
        idx = dst_v[pl.ds(i * LANES, LANES)]
        plsc.addupdate_scatter(deg_v, [idx], ones16)
        return carry

    lax.fori_loop(0, DEG_STEPS, _count, 0, unroll=4)
    pltpu.sync_copy(deg_v, out_hbm.at[wid])


def _prep_body(deg_ref, x_ref, w_ref, g_ref, dis_ref):
    deg = jnp.sum(deg_ref[...], axis=1) + 1.0
    dis = lax.rsqrt(deg)
    h = jnp.dot(x_ref[...], w_ref[...], preferred_element_type=jnp.float32)
    g_ref[...] = h * dis[:, None]
    dis_ref[...] = dis[:, None]


_BR = 2000


def _tc_prep(deg_parts, x, W):
    return pl.pallas_call(
        _prep_body,
        grid=(N // _BR,),
        in_specs=[
            pl.BlockSpec((_BR, NW), lambda i: (i, 0)),
            pl.BlockSpec((_BR, C), lambda i: (i, 0)),
            pl.BlockSpec((C, C), lambda i: (0, 0)),
        ],
        out_specs=[
            pl.BlockSpec((_BR, C), lambda i: (i, 0)),
            pl.BlockSpec((_BR, 1), lambda i: (i, 0)),
        ],
        out_shape=[
            jax.ShapeDtypeStruct((N, C), jnp.float32),
            jax.ShapeDtypeStruct((N, 1), jnp.float32),
        ],
    )(deg_parts, x, W)


@functools.partial(
    pl.kernel,
    out_type=jax.ShapeDtypeStruct((NC, N_ACC, C), jnp.float32),
    mesh=_mesh,
    compiler_params=pltpu.CompilerParams(needs_layout_passes=False),
    scratch_types=[
        pltpu.VMEM((2, BLK, 2, CHUNK), jnp.int32),
        pltpu.VMEM((NBUF, CHUNK, C), jnp.float32),
        pltpu.VMEM_SHARED((N_ACC, C), jnp.float32),
        [pltpu.SemaphoreType.DMA] * NBUF,
        [pltpu.SemaphoreType.DMA] * NBUF,
        [pltpu.SemaphoreType.DMA] * 2,
    ],
)
def _scatter_kernel(g_hbm, idx_hbm, zero_hbm, out_hbm,
                    idx_v, rows_v, acc_sh, gsems, ssems, isems):
    cid = lax.axis_index("c")
    sid = lax.axis_index("s")
    wid = cid * NS + sid

    stripe = pl.ds(sid * RPW, RPW)
    tail = pl.ds(NS * RPW, RTAIL)
    pltpu.sync_copy(zero_hbm.at[stripe], acc_sh.at[stripe])

    @pl.when(sid == NS - 1)
    def _zero_tail():
        pltpu.sync_copy(zero_hbm.at[tail], acc_sh.at[tail])

    my_idx = idx_hbm.at[wid]
    pltpu.sync_copy(my_idx.at[pl.ds(0, BLK)], idx_v.at[0])
    pltpu.async_copy(my_idx.at[pl.ds(BLK, BLK)], idx_v.at[1], isems[1])

    def _start_gather(ib, u, b):
        pltpu.async_copy(g_hbm.at[idx_v.at[ib].at[u].at[0]], rows_v.at[b],
                         gsems[b])

    def _wait_gather(b):
        pltpu.make_async_copy(g_hbm.at[idx_v.at[0].at[0].at[0]],
                              rows_v.at[b], gsems[b]).wait()

    def _start_scatter(ib, u, b):
        pltpu.async_copy(rows_v.at[b], acc_sh.at[idx_v.at[ib].at[u].at[1]],
                         ssems[b], add=True)

    def _wait_scatter(b):
        pltpu.make_async_copy(rows_v.at[b], acc_sh.at[idx_v.at[0].at[0].at[1]],
                              ssems[b]).wait()

    _start_gather(0, 0, 0)
    plsc.subcore_barrier()

    def _chunk(i, ib, u):
        b = u % NBUF
        ob = (u + NBUF - 1) % NBUF
        if u >= NBUF - 1:
            _wait_scatter(ob)
        else:

            @pl.when(i >= 1)
            def _():
                _wait_scatter(ob)

        if u == 1:
            @pl.when(jnp.logical_and(i >= 1, i + 1 <= NBLK - 1))
            def _():
                pltpu.async_copy(my_idx.at[pl.ds((i + 1) * BLK, BLK)],
                                 idx_v.at[1 - ib], isems[1 - ib])

        nb = (u + 1) % NBUF
        if u < BLK - 1:
            _start_gather(ib, u + 1, nb)
        else:

            @pl.when(i + 1 <= NBLK - 1)
            def _():
                pltpu.make_async_copy(my_idx.at[pl.ds(0, BLK)],
                                      idx_v.at[1 - ib], isems[1 - ib]).wait()
                _start_gather(1 - ib, 0, nb)

        _wait_gather(b)
        _start_scatter(ib, u, b)

    def _block_pair(ip, carry):
        for ib in (0, 1):
            i = ip * 2 + ib
            for u in range(BLK):
                _chunk(i, ib, u)
        return carry

    lax.fori_loop(0, NBLK // 2, _block_pair, 0)
    for j in range(NCHUNK - NBUF + 1, NCHUNK):
        _wait_scatter((j % BLK) % NBUF)
    plsc.subcore_barrier()
    pltpu.sync_copy(acc_sh.at[stripe], out_hbm.at[cid].at[stripe])

    @pl.when(sid == NS - 1)
    def _dump_tail():
        pltpu.sync_copy(acc_sh.at[tail], out_hbm.at[cid].at[tail])


def _final_body(sp_ref, g_ref, dis_ref, b_ref, o_ref):
    z = (sp_ref[0] + sp_ref[1] + g_ref[...]) * dis_ref[...] + b_ref[...]
    m = jnp.max(z, axis=1, keepdims=True)
    e = z - m
    o_ref[...] = e - jnp.log(jnp.sum(jnp.exp(e), axis=1, keepdims=True))


def _tc_final(s_parts, g, dis, b2d):
    return pl.pallas_call(
        _final_body,
        grid=(N // _BR,),
        in_specs=[
            pl.BlockSpec((NC, _BR, C), lambda i: (0, i, 0)),

            pl.BlockSpec((_BR, C), lambda i: (i, 0)),
            pl.BlockSpec((_BR, 1), lambda i: (i, 0)),
            pl.BlockSpec((1, C), lambda i: (0, 0)),
        ],
        out_specs=pl.BlockSpec((_BR, C), lambda i: (i, 0)),
        out_shape=jax.ShapeDtypeStruct((N, C), jnp.float32),
    )(s_parts, g, dis, b2d)


def kernel(x, edge_index, W, b):
    src = edge_index[0]
    dst = edge_index[1]
    ppt = EPT - EPW
    pad_src = jnp.zeros((NW, ppt), jnp.int32)
    sub = jnp.arange(NW, dtype=jnp.int32) % NS
    pad_dst = (N + sub[:, None] * 16
               + (jnp.arange(ppt, dtype=jnp.int32) % 16)[None, :])
    srcp = jnp.concatenate([src.reshape(NW, EPW), pad_src], axis=1)
    dstp = jnp.concatenate([dst.reshape(NW, EPW), pad_dst], axis=1)
    idx = jnp.stack(
        [srcp.reshape(NW, NCHUNK, CHUNK), dstp.reshape(NW, NCHUNK, CHUNK)],
        axis=2)
    deg_parts = _deg_kernel(dst.reshape(NW, EPW))
    g, dis = _tc_prep(deg_parts.T, x, W)
    zeros = jnp.zeros((N_ACC, C), jnp.float32)
    s_parts = _scatter_kernel(g, idx, zeros)
    return _tc_final(s_parts, g, dis, b.reshape(1, C))


# speedup vs baseline: 2.3782x; 2.3782x over previous
import functools

import jax
import jax.numpy as jnp
from jax import lax
from jax.experimental import pallas as pl
from jax.experimental.pallas import tpu as pltpu
from jax.experimental.pallas import tpu_sc as plsc

N = 10000
E = 320000
C = 128

NC = 2
NS = 16
NW = NC * NS
EPW = E // NW
LANES = 16
DEG_STEPS = EPW // LANES

CHUNK = 64
NCHUNK = 160
EPT = NCHUNK * CHUNK
E_PAD = NW * EPT
BLK = 8
NBLK = NCHUNK // BLK
NBUF = 2
N_ACC = 10256
RPW = 640
RTAIL = N_ACC - NS * RPW

_mesh = plsc.VectorSubcoreMesh(core_axis_name="c", subcore_axis_name="s")


@functools.partial(
    pl.kernel,
    out_type=jax.ShapeDtypeStruct((NW, N), jnp.float32),
    mesh=_mesh,
    compiler_params=pltpu.CompilerParams(needs_layout_passes=False),
    scratch_types=[
        pltpu.VMEM((EPW,), jnp.int32),
        pltpu.VMEM((N,), jnp.float32),
    ],
)
def _deg_kernel(dst_hbm, out_hbm, dst_v, deg_v):
    wid = lax.axis_index("c") * NS + lax.axis_index("s")
    pltpu.sync_copy(dst_hbm.at[wid], dst_v)

    zeros16 = jnp.zeros((LANES,), jnp.float32)
    ones16 = jnp.ones((LANES,), jnp.float32)

    def _zero(i, carry):
        deg_v[pl.ds(i * LANES, LANES)] = zeros16
        return carry

    lax.fori_loop(0, N // LANES, _zero, 0, unroll=8)

    def _count(i, carry):
        idx = dst_v[pl.ds(i * LANES, LANES)]
        plsc.addupdate_scatter(deg_v, [idx], ones16)
        return carry

    lax.fori_loop(0, DEG_STEPS, _count, 0, unroll=4)
    pltpu.sync_copy(deg_v, out_hbm.at[wid])


def _prep_body(deg_ref, x_ref, w_ref, g_ref, dis_ref):
    deg = jnp.sum(deg_ref[...], axis=1) + 1.0
    dis = lax.rsqrt(deg)
    h = jnp.dot(x_ref[...], w_ref[...], preferred_element_type=jnp.float32)
    g_ref[...] = h * dis[:, None]
    dis_ref[...] = dis[:, None]


_BR = 2000


def _tc_prep(deg_parts, x, W):
    return pl.pallas_call(
        _prep_body,
        grid=(N // _BR,),
        in_specs=[
            pl.BlockSpec((_BR, NW), lambda i: (i, 0)),
            pl.BlockSpec((_BR, C), lambda i: (i, 0)),
            pl.BlockSpec((C, C), lambda i: (0, 0)),
        ],
        out_specs=[
            pl.BlockSpec((_BR, C), lambda i: (i, 0)),
            pl.BlockSpec((_BR, 1), lambda i: (i, 0)),
        ],
        out_shape=[
            jax.ShapeDtypeStruct((N, C), jnp.float32),
            jax.ShapeDtypeStruct((N, 1), jnp.float32),
        ],
    )(deg_parts, x, W)


@functools.partial(
    pl.kernel,
    out_type=jax.ShapeDtypeStruct((NC, N_ACC, C), jnp.float32),
    mesh=_mesh,
    compiler_params=pltpu.CompilerParams(needs_layout_passes=False),
    scratch_types=[
        pltpu.VMEM((2, BLK, 2, CHUNK), jnp.int32),
        pltpu.VMEM((NBUF, CHUNK, C), jnp.float32),
        pltpu.VMEM_SHARED((N_ACC, C), jnp.float32),
        [pltpu.SemaphoreType.DMA] * NBUF,
        [pltpu.SemaphoreType.DMA] * NBUF,
        [pltpu.SemaphoreType.DMA] * 2,
    ],
)
def _scatter_kernel(g_hbm, idx_hbm, zero_hbm, out_hbm,
                    idx_v, rows_v, acc_sh, gsems, ssems, isems):
    cid = lax.axis_index("c")
    sid = lax.axis_index("s")
    wid = cid * NS + sid

    stripe = pl.ds(sid * RPW, RPW)
    tail = pl.ds(NS * RPW, RTAIL)
    pltpu.sync_copy(zero_hbm.at[stripe], acc_sh.at[stripe])

    @pl.when(sid == NS - 1)
    def _zero_tail():
        pltpu.sync_copy(zero_hbm.at[tail], acc_sh.at[tail])

    my_idx = idx_hbm.at[wid]
    pltpu.sync_copy(my_idx.at[pl.ds(0, BLK)], idx_v.at[0])
    pltpu.async_copy(my_idx.at[pl.ds(BLK, BLK)], idx_v.at[1], isems[1])

    def _start_gather(ib, u, b):
        pltpu.async_copy(g_hbm.at[idx_v.at[ib].at[u].at[0]], rows_v.at[b],
                         gsems[b])

    def _wait_gather(b):
        pltpu.make_async_copy(g_hbm.at[idx_v.at[0].at[0].at[0]],
                              rows_v.at[b], gsems[b]).wait()

    def _start_scatter(ib, u, b):
        pltpu.async_copy(rows_v.at[b], acc_sh.at[idx_v.at[ib].at[u].at[1]],
                         ssems[b], add=True)

    def _wait_scatter(b):
        pltpu.make_async_copy(rows_v.at[b], acc_sh.at[idx_v.at[0].at[0].at[1]],
                              ssems[b]).wait()

    _start_gather(0, 0, 0)
    plsc.subcore_barrier()

    def _chunk(i, ib, u):
        b = u % NBUF
        ob = (u + NBUF - 1) % NBUF
        if u >= NBUF - 1:
            _wait_scatter(ob)
        else:

            @pl.when(i >= 1)
            def _():
                _wait_scatter(ob)

        if u == 1:
            @pl.when(jnp.logical_and(i >= 1, i + 1 <= NBLK - 1))
            def _():
                pltpu.async_copy(my_idx.at[pl.ds((i + 1) * BLK, BLK)],
                                 idx_v.at[1 - ib], isems[1 - ib])

        nb = (u + 1) % NBUF
        if u < BLK - 1:
            _start_gather(ib, u + 1, nb)
        else:

            @pl.when(i + 1 <= NBLK - 1)
            def _():
                pltpu.make_async_copy(my_idx.at[pl.ds(0, BLK)],
                                      idx_v.at[1 - ib], isems[1 - ib]).wait()
                _start_gather(1 - ib, 0, nb)

        _wait_gather(b)
        _start_scatter(ib, u, b)

    def _block_pair(ip, carry):
        for ib in (0, 1):
            i = ip * 2 + ib
            for u in range(BLK):
                _chunk(i, ib, u)
        return carry

    lax.fori_loop(0, NBLK // 2, _block_pair, 0)
    for j in range(NCHUNK - NBUF + 1, NCHUNK):
        _wait_scatter((j % BLK) % NBUF)
    plsc.subcore_barrier()
    pltpu.sync_copy(acc_sh.at[stripe], out_hbm.at[cid].at[stripe])

    @pl.when(sid == NS - 1)
    def _dump_tail():
        pltpu.sync_copy(acc_sh.at[tail], out_hbm.at[cid].at[tail])


def _final_body(sp_ref, g_ref, dis_ref, b_ref, o_ref):
    z = (sp_ref[0] + sp_ref[1] + g_ref[...]) * dis_ref[...] + b_ref[...]
    m = jnp.max(z, axis=1, keepdims=True)
    e = z - m
    o_ref[...] = e - jnp.log(jnp.sum(jnp.exp(e), axis=1, keepdims=True))


def _tc_final(s_parts, g, dis, b2d):
    return pl.pallas_call(
        _final_body,
        grid=(N // _BR,),
        in_specs=[
            pl.BlockSpec((NC, _BR, C), lambda i: (0, i, 0)),

            pl.BlockSpec((_BR, C), lambda i: (i, 0)),
            pl.BlockSpec((_BR, 1), lambda i: (i, 0)),
            pl.BlockSpec((1, C), lambda i: (0, 0)),
        ],
        out_specs=pl.BlockSpec((_BR, C), lambda i: (i, 0)),
        out_shape=jax.ShapeDtypeStruct((N, C), jnp.float32),
    )(s_parts, g, dis, b2d)


def kernel(x, edge_index, W, b):
    src = edge_index[0]
    dst = edge_index[1]
    ppt = EPT - EPW
    pad_src = jnp.tile(jnp.arange(ppt, dtype=jnp.int32) * 41 % N, (NW, 1))
    sub = jnp.arange(NW, dtype=jnp.int32) % NS
    pad_dst = (N + sub[:, None] * 16
               + (jnp.arange(ppt, dtype=jnp.int32) % 16)[None, :])
    srcp = jnp.concatenate([src.reshape(NW, EPW), pad_src], axis=1)
    dstp = jnp.concatenate([dst.reshape(NW, EPW), pad_dst], axis=1)
    idx = jnp.stack(
        [srcp.reshape(NW, NCHUNK, CHUNK), dstp.reshape(NW, NCHUNK, CHUNK)],
        axis=2)
    deg_parts = _deg_kernel(dst.reshape(NW, EPW))
    g, dis = _tc_prep(deg_parts.T, x, W)
    zeros = jnp.zeros((N_ACC, C), jnp.float32)
    s_parts = _scatter_kernel(g, idx, zeros)
    return _tc_final(s_parts, g, dis, b.reshape(1, C))


# speedup vs baseline: 2.4726x; 1.0397x over previous
import functools

import jax
import jax.numpy as jnp
from jax import lax
from jax.experimental import pallas as pl
from jax.experimental.pallas import tpu as pltpu
from jax.experimental.pallas import tpu_sc as plsc

N = 10000
E = 320000
C = 128

NC = 2
NS = 16
NW = NC * NS
EPW = E // NW
LANES = 16
DEG_STEPS = EPW // LANES

CHUNK = 64
NCHUNK = 160
EPT = NCHUNK * CHUNK
E_PAD = NW * EPT
BLK = 8
NBLK = NCHUNK // BLK
NBUF = 4
N_ACC = 10256
RPW = 640
RTAIL = N_ACC - NS * RPW

_mesh = plsc.VectorSubcoreMesh(core_axis_name="c", subcore_axis_name="s")


@functools.partial(
    pl.kernel,
    out_type=jax.ShapeDtypeStruct((NW, N), jnp.float32),
    mesh=_mesh,
    compiler_params=pltpu.CompilerParams(needs_layout_passes=False),
    scratch_types=[
        pltpu.VMEM((EPW,), jnp.int32),
        pltpu.VMEM((N,), jnp.float32),
    ],
)
def _deg_kernel(dst_hbm, out_hbm, dst_v, deg_v):
    wid = lax.axis_index("c") * NS + lax.axis_index("s")
    pltpu.sync_copy(dst_hbm.at[wid], dst_v)

    zeros16 = jnp.zeros((LANES,), jnp.float32)
    ones16 = jnp.ones((LANES,), jnp.float32)

    def _zero(i, carry):
        deg_v[pl.ds(i * LANES, LANES)] = zeros16
        return carry

    lax.fori_loop(0, N // LANES, _zero, 0, unroll=8)

    def _count(i, carry):
        idx = dst_v[pl.ds(i * LANES, LANES)]
        plsc.addupdate_scatter(deg_v, [idx], ones16)
        return carry

    lax.fori_loop(0, DEG_STEPS, _count, 0, unroll=4)
    pltpu.sync_copy(deg_v, out_hbm.at[wid])


def _prep_body(deg_ref, x_ref, w_ref, g_ref, dis_ref):
    deg = jnp.sum(deg_ref[...], axis=1) + 1.0
    dis = lax.rsqrt(deg)
    h = jnp.dot(x_ref[...], w_ref[...], preferred_element_type=jnp.float32)
    g_ref[...] = h * dis[:, None]
    dis_ref[...] = dis[:, None]


_BR = 2000


def _tc_prep(deg_parts, x, W):
    return pl.pallas_call(
        _prep_body,
        grid=(N // _BR,),
        in_specs=[
            pl.BlockSpec((_BR, NW), lambda i: (i, 0)),
            pl.BlockSpec((_BR, C), lambda i: (i, 0)),
            pl.BlockSpec((C, C), lambda i: (0, 0)),
        ],
        out_specs=[
            pl.BlockSpec((_BR, C), lambda i: (i, 0)),
            pl.BlockSpec((_BR, 1), lambda i: (i, 0)),
        ],
        out_shape=[
            jax.ShapeDtypeStruct((N, C), jnp.float32),
            jax.ShapeDtypeStruct((N, 1), jnp.float32),
        ],
    )(deg_parts, x, W)


@functools.partial(
    pl.kernel,
    out_type=jax.ShapeDtypeStruct((NC, N_ACC, C), jnp.float32),
    mesh=_mesh,
    compiler_params=pltpu.CompilerParams(needs_layout_passes=False),
    scratch_types=[
        pltpu.VMEM((2, BLK, 2, CHUNK), jnp.int32),
        pltpu.VMEM((NBUF, CHUNK, C), jnp.float32),
        pltpu.VMEM_SHARED((N_ACC, C), jnp.float32),
        [pltpu.SemaphoreType.DMA] * NBUF,
        [pltpu.SemaphoreType.DMA] * NBUF,
        [pltpu.SemaphoreType.DMA] * 2,
    ],
)
def _scatter_kernel(g_hbm, idx_hbm, zero_hbm, out_hbm,
                    idx_v, rows_v, acc_sh, gsems, ssems, isems):
    cid = lax.axis_index("c")
    sid = lax.axis_index("s")
    wid = cid * NS + sid

    stripe = pl.ds(sid * RPW, RPW)
    tail = pl.ds(NS * RPW, RTAIL)
    pltpu.sync_copy(zero_hbm.at[stripe], acc_sh.at[stripe])

    @pl.when(sid == NS - 1)
    def _zero_tail():
        pltpu.sync_copy(zero_hbm.at[tail], acc_sh.at[tail])

    my_idx = idx_hbm.at[wid]
    pltpu.sync_copy(my_idx.at[pl.ds(0, BLK)], idx_v.at[0])
    pltpu.async_copy(my_idx.at[pl.ds(BLK, BLK)], idx_v.at[1], isems[1])

    def _start_gather(ib, u, b):
        pltpu.async_copy(g_hbm.at[idx_v.at[ib].at[u].at[0]], rows_v.at[b],
                         gsems[b])

    def _wait_gather(b):
        pltpu.make_async_copy(g_hbm.at[idx_v.at[0].at[0].at[0]],
                              rows_v.at[b], gsems[b]).wait()

    def _start_scatter(ib, u, b):
        pltpu.async_copy(rows_v.at[b], acc_sh.at[idx_v.at[ib].at[u].at[1]],
                         ssems[b], add=True)

    def _wait_scatter(b):
        pltpu.make_async_copy(rows_v.at[b], acc_sh.at[idx_v.at[0].at[0].at[1]],
                              ssems[b]).wait()

    _start_gather(0, 0, 0)
    plsc.subcore_barrier()

    def _chunk(i, ib, u):
        b = u % NBUF
        nb = (u + 1) % NBUF
        if u >= NBUF - 1:
            _wait_scatter(nb)
        else:

            @pl.when(i >= 1)
            def _():
                _wait_scatter(nb)

        if u == 1:
            @pl.when(jnp.logical_and(i >= 1, i + 1 <= NBLK - 1))
            def _():
                pltpu.async_copy(my_idx.at[pl.ds((i + 1) * BLK, BLK)],
                                 idx_v.at[1 - ib], isems[1 - ib])

        if u < BLK - 1:
            _start_gather(ib, u + 1, nb)
        else:

            @pl.when(i + 1 <= NBLK - 1)
            def _():
                pltpu.make_async_copy(my_idx.at[pl.ds(0, BLK)],
                                      idx_v.at[1 - ib], isems[1 - ib]).wait()
                _start_gather(1 - ib, 0, nb)

        _wait_gather(b)
        _start_scatter(ib, u, b)

    def _block_pair(ip, carry):
        for ib in (0, 1):
            i = ip * 2 + ib
            for u in range(BLK):
                _chunk(i, ib, u)
        return carry

    lax.fori_loop(0, NBLK // 2, _block_pair, 0)
    for j in range(NCHUNK - NBUF + 1, NCHUNK):
        _wait_scatter((j % BLK) % NBUF)
    plsc.subcore_barrier()
    pltpu.sync_copy(acc_sh.at[stripe], out_hbm.at[cid].at[stripe])

    @pl.when(sid == NS - 1)
    def _dump_tail():
        pltpu.sync_copy(acc_sh.at[tail], out_hbm.at[cid].at[tail])


def _final_body(sp_ref, g_ref, dis_ref, b_ref, o_ref):
    z = (sp_ref[0] + sp_ref[1] + g_ref[...]) * dis_ref[...] + b_ref[...]
    m = jnp.max(z, axis=1, keepdims=True)
    e = z - m
    o_ref[...] = e - jnp.log(jnp.sum(jnp.exp(e), axis=1, keepdims=True))


def _tc_final(s_parts, g, dis, b2d):
    return pl.pallas_call(
        _final_body,
        grid=(N // _BR,),
        in_specs=[
            pl.BlockSpec((NC, _BR, C), lambda i: (0, i, 0)),

            pl.BlockSpec((_BR, C), lambda i: (i, 0)),
            pl.BlockSpec((_BR, 1), lambda i: (i, 0)),
            pl.BlockSpec((1, C), lambda i: (0, 0)),
        ],
        out_specs=pl.BlockSpec((_BR, C), lambda i: (i, 0)),
        out_shape=jax.ShapeDtypeStruct((N, C), jnp.float32),
    )(s_parts, g, dis, b2d)


def kernel(x, edge_index, W, b):
    src = edge_index[0]
    dst = edge_index[1]
    ppt = EPT - EPW
    pad_src = jnp.tile(jnp.arange(ppt, dtype=jnp.int32) * 41 % N, (NW, 1))
    sub = jnp.arange(NW, dtype=jnp.int32) % NS
    pad_dst = (N + sub[:, None] * 16
               + (jnp.arange(ppt, dtype=jnp.int32) % 16)[None, :])
    srcp = jnp.concatenate([src.reshape(NW, EPW), pad_src], axis=1)
    dstp = jnp.concatenate([dst.reshape(NW, EPW), pad_dst], axis=1)
    idx = jnp.stack(
        [srcp.reshape(NW, NCHUNK, CHUNK), dstp.reshape(NW, NCHUNK, CHUNK)],
        axis=2)
    deg_parts = _deg_kernel(dst.reshape(NW, EPW))
    g, dis = _tc_prep(deg_parts.T, x, W)
    zeros = jnp.zeros((N_ACC, C), jnp.float32)
    s_parts = _scatter_kernel(g, idx, zeros)
    return _tc_final(s_parts, g, dis, b.reshape(1, C))


# speedup vs baseline: 2.7712x; 1.1208x over previous
import functools

import jax
import jax.numpy as jnp
from jax import lax
from jax.experimental import pallas as pl
from jax.experimental.pallas import tpu as pltpu
from jax.experimental.pallas import tpu_sc as plsc

N = 10000
E = 320000
C = 128

NC = 2
NS = 16
NW = NC * NS
EPW = E // NW
LANES = 16
DEG_STEPS = EPW // LANES

CHUNK = 128
NCHUNK = 80
EPT = NCHUNK * CHUNK
E_PAD = NW * EPT
BLK = 4
NBLK = NCHUNK // BLK
NBUF = 2
N_ACC = 10256
RPW = 640
RTAIL = N_ACC - NS * RPW

_mesh = plsc.VectorSubcoreMesh(core_axis_name="c", subcore_axis_name="s")


@functools.partial(
    pl.kernel,
    out_type=jax.ShapeDtypeStruct((NW, N), jnp.float32),
    mesh=_mesh,
    compiler_params=pltpu.CompilerParams(needs_layout_passes=False),
    scratch_types=[
        pltpu.VMEM((EPW,), jnp.int32),
        pltpu.VMEM((N,), jnp.float32),
    ],
)
def _deg_kernel(dst_hbm, out_hbm, dst_v, deg_v):
    wid = lax.axis_index("c") * NS + lax.axis_index("s")
    pltpu.sync_copy(dst_hbm.at[wid], dst_v)

    zeros16 = jnp.zeros((LANES,), jnp.float32)
    ones16 = jnp.ones((LANES,), jnp.float32)

    def _zero(i, carry):
        deg_v[pl.ds(i * LANES, LANES)] = zeros16
        return carry

    lax.fori_loop(0, N // LANES, _zero, 0, unroll=8)

    def _count(i, carry):
        idx = dst_v[pl.ds(i * LANES, LANES)]
        plsc.addupdate_scatter(deg_v, [idx], ones16)
        return carry

    lax.fori_loop(0, DEG_STEPS, _count, 0, unroll=4)
    pltpu.sync_copy(deg_v, out_hbm.at[wid])


def _prep_body(deg_ref, x_ref, w_ref, g_ref, dis_ref):
    deg = jnp.sum(deg_ref[...], axis=1) + 1.0
    dis = lax.rsqrt(deg)
    h = jnp.dot(x_ref[...], w_ref[...], preferred_element_type=jnp.float32)
    g_ref[...] = h * dis[:, None]
    dis_ref[...] = dis[:, None]


_BR = 2000


def _tc_prep(deg_parts, x, W):
    return pl.pallas_call(
        _prep_body,
        grid=(N // _BR,),
        in_specs=[
            pl.BlockSpec((_BR, NW), lambda i: (i, 0)),
            pl.BlockSpec((_BR, C), lambda i: (i, 0)),
            pl.BlockSpec((C, C), lambda i: (0, 0)),
        ],
        out_specs=[
            pl.BlockSpec((_BR, C), lambda i: (i, 0)),
            pl.BlockSpec((_BR, 1), lambda i: (i, 0)),
        ],
        out_shape=[
            jax.ShapeDtypeStruct((N, C), jnp.float32),
            jax.ShapeDtypeStruct((N, 1), jnp.float32),
        ],
    )(deg_parts, x, W)


@functools.partial(
    pl.kernel,
    out_type=jax.ShapeDtypeStruct((NC, N_ACC, C), jnp.float32),
    mesh=_mesh,
    compiler_params=pltpu.CompilerParams(needs_layout_passes=False),
    scratch_types=[
        pltpu.VMEM((2, BLK, 2, CHUNK), jnp.int32),
        pltpu.VMEM((NBUF, CHUNK, C), jnp.float32),
        pltpu.VMEM_SHARED((N_ACC, C), jnp.float32),
        [pltpu.SemaphoreType.DMA] * NBUF,
        [pltpu.SemaphoreType.DMA] * NBUF,
        [pltpu.SemaphoreType.DMA] * 2,
    ],
)
def _scatter_kernel(g_hbm, idx_hbm, zero_hbm, out_hbm,
                    idx_v, rows_v, acc_sh, gsems, ssems, isems):
    cid = lax.axis_index("c")
    sid = lax.axis_index("s")
    wid = cid * NS + sid

    stripe = pl.ds(sid * RPW, RPW)
    tail = pl.ds(NS * RPW, RTAIL)
    pltpu.sync_copy(zero_hbm.at[stripe], acc_sh.at[stripe])

    @pl.when(sid == NS - 1)
    def _zero_tail():
        pltpu.sync_copy(zero_hbm.at[tail], acc_sh.at[tail])

    my_idx = idx_hbm.at[wid]
    pltpu.sync_copy(my_idx.at[pl.ds(0, BLK)], idx_v.at[0])
    pltpu.async_copy(my_idx.at[pl.ds(BLK, BLK)], idx_v.at[1], isems[1])

    def _start_gather(ib, u, b):
        pltpu.async_copy(g_hbm.at[idx_v.at[ib].at[u].at[0]], rows_v.at[b],
                         gsems[b])

    def _wait_gather(b):
        pltpu.make_async_copy(g_hbm.at[idx_v.at[0].at[0].at[0]],
                              rows_v.at[b], gsems[b]).wait()

    def _start_scatter(ib, u, b):
        pltpu.async_copy(rows_v.at[b], acc_sh.at[idx_v.at[ib].at[u].at[1]],
                         ssems[b], add=True)

    def _wait_scatter(b):
        pltpu.make_async_copy(rows_v.at[b], acc_sh.at[idx_v.at[0].at[0].at[1]],
                              ssems[b]).wait()

    _start_gather(0, 0, 0)
    plsc.subcore_barrier()

    def _chunk(i, ib, u):
        b = u % NBUF
        nb = (u + 1) % NBUF
        if u >= NBUF - 1:
            _wait_scatter(nb)
        else:

            @pl.when(i >= 1)
            def _():
                _wait_scatter(nb)

        if u == 1:
            @pl.when(jnp.logical_and(i >= 1, i + 1 <= NBLK - 1))
            def _():
                pltpu.async_copy(my_idx.at[pl.ds((i + 1) * BLK, BLK)],
                                 idx_v.at[1 - ib], isems[1 - ib])

        if u < BLK - 1:
            _start_gather(ib, u + 1, nb)
        else:

            @pl.when(i + 1 <= NBLK - 1)
            def _():
                pltpu.make_async_copy(my_idx.at[pl.ds(0, BLK)],
                                      idx_v.at[1 - ib], isems[1 - ib]).wait()
                _start_gather(1 - ib, 0, nb)

        _wait_gather(b)
        _start_scatter(ib, u, b)

    def _block_pair(ip, carry):
        for ib in (0, 1):
            i = ip * 2 + ib
            for u in range(BLK):
                _chunk(i, ib, u)
        return carry

    lax.fori_loop(0, NBLK // 2, _block_pair, 0)
    for j in range(NCHUNK - NBUF + 1, NCHUNK):
        _wait_scatter((j % BLK) % NBUF)
    plsc.subcore_barrier()
    pltpu.sync_copy(acc_sh.at[stripe], out_hbm.at[cid].at[stripe])

    @pl.when(sid == NS - 1)
    def _dump_tail():
        pltpu.sync_copy(acc_sh.at[tail], out_hbm.at[cid].at[tail])


def _final_body(sp_ref, g_ref, dis_ref, b_ref, o_ref):
    z = (sp_ref[0] + sp_ref[1] + g_ref[...]) * dis_ref[...] + b_ref[...]
    m = jnp.max(z, axis=1, keepdims=True)
    e = z - m
    o_ref[...] = e - jnp.log(jnp.sum(jnp.exp(e), axis=1, keepdims=True))


def _tc_final(s_parts, g, dis, b2d):
    return pl.pallas_call(
        _final_body,
        grid=(N // _BR,),
        in_specs=[
            pl.BlockSpec((NC, _BR, C), lambda i: (0, i, 0)),

            pl.BlockSpec((_BR, C), lambda i: (i, 0)),
            pl.BlockSpec((_BR, 1), lambda i: (i, 0)),
            pl.BlockSpec((1, C), lambda i: (0, 0)),
        ],
        out_specs=pl.BlockSpec((_BR, C), lambda i: (i, 0)),
        out_shape=jax.ShapeDtypeStruct((N, C), jnp.float32),
    )(s_parts, g, dis, b2d)


def kernel(x, edge_index, W, b):
    src = edge_index[0]
    dst = edge_index[1]
    ppt = EPT - EPW
    pad_src = jnp.tile(jnp.arange(ppt, dtype=jnp.int32) * 41 % N, (NW, 1))
    sub = jnp.arange(NW, dtype=jnp.int32) % NS
    pad_dst = (N + sub[:, None] * 16
               + (jnp.arange(ppt, dtype=jnp.int32) % 16)[None, :])
    srcp = jnp.concatenate([src.reshape(NW, EPW), pad_src], axis=1)
    dstp = jnp.concatenate([dst.reshape(NW, EPW), pad_dst], axis=1)
    idx = jnp.stack(
        [srcp.reshape(NW, NCHUNK, CHUNK), dstp.reshape(NW, NCHUNK, CHUNK)],
        axis=2)
    deg_parts = _deg_kernel(dst.reshape(NW, EPW))
    g, dis = _tc_prep(deg_parts.T, x, W)
    zeros = jnp.zeros((N_ACC, C), jnp.float32)
    s_parts = _scatter_kernel(g, idx, zeros)
    return _tc_final(s_parts, g, dis, b.reshape(1, C))


# speedup vs baseline: 2.9450x; 1.0627x over previous
import functools

import jax
import jax.numpy as jnp
from jax import lax
from jax.experimental import pallas as pl
from jax.experimental.pallas import tpu as pltpu
from jax.experimental.pallas import tpu_sc as plsc

N = 10000
E = 320000
C = 128

NC = 2
NS = 16
NW = NC * NS
EPW = E // NW
LANES = 16
DEG_STEPS = EPW // LANES

CHUNK = 128
NCHUNK = 80
EPT = NCHUNK * CHUNK
E_PAD = NW * EPT
BLK = 4
NBLK = NCHUNK // BLK
NBUF = 2
N_ACC = 10256
RPW = 640
RTAIL = N_ACC - NS * RPW

_mesh = plsc.VectorSubcoreMesh(core_axis_name="c", subcore_axis_name="s")


@functools.partial(
    pl.kernel,
    out_type=jax.ShapeDtypeStruct((NW, N), jnp.float32),
    mesh=_mesh,
    compiler_params=pltpu.CompilerParams(needs_layout_passes=False),
    scratch_types=[
        pltpu.VMEM((EPW,), jnp.int32),
        pltpu.VMEM((N,), jnp.float32),
    ],
)
def _deg_kernel(dst_hbm, out_hbm, dst_v, deg_v):
    wid = lax.axis_index("c") * NS + lax.axis_index("s")
    pltpu.sync_copy(dst_hbm.at[wid], dst_v)

    zeros16 = jnp.zeros((LANES,), jnp.float32)
    ones16 = jnp.ones((LANES,), jnp.float32)

    def _zero(i, carry):
        deg_v[pl.ds(i * LANES, LANES)] = zeros16
        return carry

    lax.fori_loop(0, N // LANES, _zero, 0, unroll=8)

    def _count(i, carry):
        idx = dst_v[pl.ds(i * LANES, LANES)]
        plsc.addupdate_scatter(deg_v, [idx], ones16)
        return carry

    lax.fori_loop(0, DEG_STEPS, _count, 0, unroll=4)
    pltpu.sync_copy(deg_v, out_hbm.at[wid])


def _prep_body(deg_ref, x_ref, w_ref, g_ref, dis_ref):
    deg = jnp.sum(deg_ref[...], axis=0) + 1.0
    dis = lax.rsqrt(deg)
    h = jnp.dot(x_ref[...], w_ref[...], preferred_element_type=jnp.float32)
    g_ref[...] = h * dis[:, None]
    dis_ref[...] = dis[:, None]


_BR = 2000


def _tc_prep(deg_parts, x, W):
    return pl.pallas_call(
        _prep_body,
        out_shape=[
            jax.ShapeDtypeStruct((N, C), jnp.float32),
            jax.ShapeDtypeStruct((N, 1), jnp.float32),
        ],
    )(deg_parts, x, W)


@functools.partial(
    pl.kernel,
    out_type=jax.ShapeDtypeStruct((NC, N_ACC, C), jnp.float32),
    mesh=_mesh,
    compiler_params=pltpu.CompilerParams(needs_layout_passes=False),
    scratch_types=[
        pltpu.VMEM((2, BLK, 2, CHUNK), jnp.int32),
        pltpu.VMEM((NBUF, CHUNK, C), jnp.float32),
        pltpu.VMEM_SHARED((N_ACC, C), jnp.float32),
        [pltpu.SemaphoreType.DMA] * NBUF,
        [pltpu.SemaphoreType.DMA] * NBUF,
        [pltpu.SemaphoreType.DMA] * 2,
    ],
)
def _scatter_kernel(g_hbm, idx_hbm, out_hbm,
                    idx_v, rows_v, acc_sh, gsems, ssems, isems):
    cid = lax.axis_index("c")
    sid = lax.axis_index("s")
    wid = cid * NS + sid

    zbuf = rows_v.at[0]
    z16 = jnp.zeros((LANES,), jnp.float32)

    def _zrow(r, carry):
        for k in range(C // LANES):
            zbuf[r, pl.ds(k * LANES, LANES)] = z16
        return carry

    lax.fori_loop(0, CHUNK, _zrow, 0)
    stripe = pl.ds(sid * RPW, RPW)
    tail = pl.ds(NS * RPW, RTAIL)
    for k in range(RPW // CHUNK):
        pltpu.sync_copy(zbuf, acc_sh.at[pl.ds(sid * RPW + k * CHUNK, CHUNK)])

    @pl.when(sid == NS - 1)
    def _zero_tail():
        pltpu.sync_copy(zbuf.at[pl.ds(0, RTAIL)], acc_sh.at[tail])

    my_idx = idx_hbm.at[wid]
    pltpu.sync_copy(my_idx.at[pl.ds(0, BLK)], idx_v.at[0])
    pltpu.async_copy(my_idx.at[pl.ds(BLK, BLK)], idx_v.at[1], isems[1])

    def _start_gather(ib, u, b):
        pltpu.async_copy(g_hbm.at[idx_v.at[ib].at[u].at[0]], rows_v.at[b],
                         gsems[b])

    def _wait_gather(b):
        pltpu.make_async_copy(g_hbm.at[idx_v.at[0].at[0].at[0]],
                              rows_v.at[b], gsems[b]).wait()

    def _start_scatter(ib, u, b):
        pltpu.async_copy(rows_v.at[b], acc_sh.at[idx_v.at[ib].at[u].at[1]],
                         ssems[b], add=True)

    def _wait_scatter(b):
        pltpu.make_async_copy(rows_v.at[b], acc_sh.at[idx_v.at[0].at[0].at[1]],
                              ssems[b]).wait()

    _start_gather(0, 0, 0)
    plsc.subcore_barrier()

    def _chunk(i, ib, u):
        b = u % NBUF
        nb = (u + 1) % NBUF
        if u >= NBUF - 1:
            _wait_scatter(nb)
        else:

            @pl.when(i >= 1)
            def _():
                _wait_scatter(nb)

        if u == 1:
            @pl.when(jnp.logical_and(i >= 1, i + 1 <= NBLK - 1))
            def _():
                pltpu.async_copy(my_idx.at[pl.ds((i + 1) * BLK, BLK)],
                                 idx_v.at[1 - ib], isems[1 - ib])

        if u < BLK - 1:
            _start_gather(ib, u + 1, nb)
        else:

            @pl.when(i + 1 <= NBLK - 1)
            def _():
                pltpu.make_async_copy(my_idx.at[pl.ds(0, BLK)],
                                      idx_v.at[1 - ib], isems[1 - ib]).wait()
                _start_gather(1 - ib, 0, nb)

        _wait_gather(b)
        _start_scatter(ib, u, b)

    def _block_pair(ip, carry):
        for ib in (0, 1):
            i = ip * 2 + ib
            for u in range(BLK):
                _chunk(i, ib, u)
        return carry

    lax.fori_loop(0, NBLK // 2, _block_pair, 0)
    for j in range(NCHUNK - NBUF + 1, NCHUNK):
        _wait_scatter((j % BLK) % NBUF)
    plsc.subcore_barrier()
    pltpu.sync_copy(acc_sh.at[stripe], out_hbm.at[cid].at[stripe])

    @pl.when(sid == NS - 1)
    def _dump_tail():
        pltpu.sync_copy(acc_sh.at[tail], out_hbm.at[cid].at[tail])


def _final_body(sp_ref, g_ref, dis_ref, b_ref, o_ref):
    z = (sp_ref[0] + sp_ref[1] + g_ref[...]) * dis_ref[...] + b_ref[...]
    m = jnp.max(z, axis=1, keepdims=True)
    e = z - m
    o_ref[...] = e - jnp.log(jnp.sum(jnp.exp(e), axis=1, keepdims=True))


def _tc_final(s_parts, g, dis, b2d):
    return pl.pallas_call(
        _final_body,
        grid=(N // _BR,),
        in_specs=[
            pl.BlockSpec((NC, _BR, C), lambda i: (0, i, 0)),

            pl.BlockSpec((_BR, C), lambda i: (i, 0)),
            pl.BlockSpec((_BR, 1), lambda i: (i, 0)),
            pl.BlockSpec((1, C), lambda i: (0, 0)),
        ],
        out_specs=pl.BlockSpec((_BR, C), lambda i: (i, 0)),
        out_shape=jax.ShapeDtypeStruct((N, C), jnp.float32),
    )(s_parts, g, dis, b2d)


def kernel(x, edge_index, W, b):
    src = edge_index[0]
    dst = edge_index[1]
    ppt = EPT - EPW
    pad_src = jnp.tile(jnp.arange(ppt, dtype=jnp.int32) * 41 % N, (NW, 1))
    sub = jnp.arange(NW, dtype=jnp.int32) % NS
    pad_dst = (N + sub[:, None] * 16
               + (jnp.arange(ppt, dtype=jnp.int32) % 16)[None, :])
    srcp = jnp.concatenate([src.reshape(NW, EPW), pad_src], axis=1)
    dstp = jnp.concatenate([dst.reshape(NW, EPW), pad_dst], axis=1)
    idx = jnp.stack(
        [srcp.reshape(NW, NCHUNK, CHUNK), dstp.reshape(NW, NCHUNK, CHUNK)],
        axis=2)
    deg_parts = _deg_kernel(dst.reshape(NW, EPW))
    g, dis = _tc_prep(deg_parts, x, W)
    s_parts = _scatter_kernel(g, idx)
    return _tc_final(s_parts, g, dis, b.reshape(1, C))
